# baseline (speedup 1.0000x reference)
import jax
import jax.numpy as jnp
from jax import lax
from jax.experimental import pallas as pl
from jax.experimental.pallas import tpu as pltpu
from jax.experimental.pallas import tpu_sc as plsc

B, L, D = 4096, 200, 64
V = 1_000_000
B_BLKS = B // 128
N = B * L
NC, NS = 2, 16
NW = NC * NS
ROWS_PER_W = N // NW
GATHER = 128
N_ITERS = ROWS_PER_W // GATHER
IDX_ROWS = ROWS_PER_W // GATHER
VB = 512
PBLKS = 977
OFFSET = PBLKS * VB
TDIMS = (((0,), (0,)), ((), ()))


def _pack_body(wa_ref, wb_ref, eye_ref, o_ref):
    ta = lax.dot_general(wa_ref[...], eye_ref[...], TDIMS,
                         preferred_element_type=jnp.float32)
    tb = lax.dot_general(wb_ref[...], eye_ref[...], TDIMS,
                         preferred_element_type=jnp.float32)
    o_ref[...] = jnp.concatenate([ta, tb], axis=1)


def _gather_body(idx_hbm, table_hbm, out_hbm, idx_v, sidx, rows_v,
                 g_sem0, g_sem1, s_sem0, s_sem1):
    wid = lax.axis_index("s") * NC + lax.axis_index("c")
    out_base = wid * ROWS_PER_W
    g_sems = (g_sem0, g_sem1)
    s_sems = (s_sem0, s_sem1)

    def make_sidx(t, buf):
        for k in range(8):
            v = idx_v[t, pl.ds(16 * k, 16)]
            sidx[buf, pl.ds(16 * k, 16)] = jnp.where(v >= OFFSET,
                                                     v - OFFSET, v)

    def issue_gather(t, buf):
        make_sidx(t, buf)
        pltpu.async_copy(table_hbm.at[sidx.at[buf]],
                         rows_v.at[buf], g_sems[buf])

    def wait_gather(buf):
        pltpu.make_async_copy(table_hbm.at[sidx.at[buf]],
                              rows_v.at[buf], g_sems[buf]).wait()

    def issue_store(t, buf):
        pltpu.async_copy(rows_v.at[buf],
                         out_hbm.at[pl.ds(out_base + t * GATHER, GATHER)],
                         s_sems[buf])

    def wait_store(buf):
        pltpu.make_async_copy(rows_v.at[buf],
                              out_hbm.at[pl.ds(out_base, GATHER)],
                              s_sems[buf]).wait()

    pltpu.sync_copy(idx_hbm.at[pl.ds(wid * IDX_ROWS, IDX_ROWS)], idx_v)

    issue_gather(0, 0)
    issue_gather(1, 1)

    def body(tt, carry):
        t0 = tt * 2
        t1 = t0 + 1
        wait_gather(0)
        issue_store(t0 - 2, 0)
        wait_gather(1)
        issue_store(t1 - 2, 1)
        wait_store(0)
        issue_gather(t0, 0)
        wait_store(1)
        issue_gather(t1, 1)
        return carry

    lax.fori_loop(1, N_ITERS // 2, body, 0)

    wait_gather(0)
    issue_store(N_ITERS - 2, 0)
    wait_gather(1)
    issue_store(N_ITERS - 1, 1)
    wait_store(0)
    wait_store(1)


def _untangle_body(g_ref, i_ref, eye_ref, o_ref):
    blk = g_ref[...]
    t = lax.dot_general(blk, eye_ref[...], TDIMS,
                        preferred_element_type=jnp.float32)
    row = i_ref[0, pl.program_id(1)]
    hi = row >= OFFSET
    o_ref[0, :, 0] = jnp.where(hi[None, :], t[64:], t[:64]).reshape(8, 8, 128)


def kernel(x, W):
    idx = x.T.reshape(N // GATHER, GATHER).astype(jnp.int32)
    wt = W.T
    eye64 = jnp.eye(64, dtype=jnp.float32)
    eye128 = jnp.eye(128, dtype=jnp.float32)

    table = pl.pallas_call(
        _pack_body,
        grid=(PBLKS,),
        in_specs=[
            pl.BlockSpec((D, VB), lambda b: (0, b)),
            pl.BlockSpec((D, VB), lambda b: (0, PBLKS + b)),
            pl.BlockSpec((D, D), lambda b: (0, 0)),
        ],
        out_specs=pl.BlockSpec((VB, 128), lambda b: (b, 0)),
        out_shape=jax.ShapeDtypeStruct((OFFSET, 128), jnp.float32),
    )(wt, wt, eye64)

    mesh = plsc.VectorSubcoreMesh(core_axis_name="c", subcore_axis_name="s")
    run = pl.kernel(
        _gather_body,
        out_type=jax.ShapeDtypeStruct((N, 128), jnp.float32),
        mesh=mesh,
        scratch_types=[
            pltpu.VMEM((IDX_ROWS, GATHER), jnp.int32),
            pltpu.VMEM((2, GATHER), jnp.int32),
            pltpu.VMEM((2, GATHER, 128), jnp.float32),
            pltpu.SemaphoreType.DMA,
            pltpu.SemaphoreType.DMA,
            pltpu.SemaphoreType.DMA,
            pltpu.SemaphoreType.DMA,
        ],
        compiler_params=pltpu.CompilerParams(use_tc_tiling_on_sc=True),
    )
    g = run(idx, table)

    out5 = pl.pallas_call(
        _untangle_body,
        grid=(L, B_BLKS),
        in_specs=[
            pl.BlockSpec((128, 128), lambda l, b: (l * B_BLKS + b, 0)),
            pl.BlockSpec((1, B_BLKS, 128), lambda l, b: (l, 0, 0)),
            pl.BlockSpec((128, 128), lambda l, b: (0, 0)),
        ],
        out_specs=pl.BlockSpec((1, 8, 1, 8, 128), lambda l, b: (l, 0, b, 0, 0)),
        out_shape=jax.ShapeDtypeStruct((L, 8, B_BLKS, 8, 128), jnp.float32),
    )(g, idx.reshape(L, B_BLKS, 128), eye128)

    return out5.transpose(2, 4, 0, 1, 3).reshape(B, L, D)


# speedup vs baseline: 3.3512x; 3.3512x over previous
import jax
import jax.numpy as jnp
from jax import lax
from jax.experimental import pallas as pl
from jax.experimental.pallas import tpu as pltpu
from jax.experimental.pallas import tpu_sc as plsc

B, L, D = 4096, 200, 64
V = 1_000_000
B_BLKS = B // 128
N = B * L
NC, NS = 2, 16
NW = NC * NS
ROWS_PER_W = N // NW
GATHER = 128
N_ITERS = ROWS_PER_W // GATHER
IDX_ROWS = ROWS_PER_W // GATHER
VB = 512
PBLKS = 977
OFFSET = PBLKS * VB
TDIMS = (((0,), (0,)), ((), ()))


def _pack_body(wa_ref, wb_ref, eye_ref, o_ref):
    ta = lax.dot_general(wa_ref[...], eye_ref[...], TDIMS,
                         preferred_element_type=jnp.float32)
    tb = lax.dot_general(wb_ref[...], eye_ref[...], TDIMS,
                         preferred_element_type=jnp.float32)
    o_ref[...] = jnp.concatenate([ta, tb], axis=1)


def _gather_body(idx_hbm, table_hbm, out_hbm, idx_v, sidx, rows_v,
                 g_sem0, g_sem1, s_sem0, s_sem1):
    wid = lax.axis_index("s") * NC + lax.axis_index("c")
    out_base = wid * ROWS_PER_W
    g_sems = (g_sem0, g_sem1)
    s_sems = (s_sem0, s_sem1)

    def make_sidx(t, buf):
        for k in range(8):
            v = idx_v[t, pl.ds(16 * k, 16)]
            sidx[buf, pl.ds(16 * k, 16)] = jnp.where(v >= OFFSET,
                                                     v - OFFSET, v)

    def issue_gather(t, buf):
        make_sidx(t, buf)
        pltpu.async_copy(table_hbm.at[sidx.at[buf]],
                         rows_v.at[buf], g_sems[buf])

    def wait_gather(buf):
        pltpu.make_async_copy(table_hbm.at[sidx.at[buf]],
                              rows_v.at[buf], g_sems[buf]).wait()

    def issue_store(t, buf):
        pltpu.async_copy(rows_v.at[buf],
                         out_hbm.at[pl.ds(out_base + t * GATHER, GATHER)],
                         s_sems[buf])

    def wait_store(buf):
        pltpu.make_async_copy(rows_v.at[buf],
                              out_hbm.at[pl.ds(out_base, GATHER)],
                              s_sems[buf]).wait()

    pltpu.sync_copy(idx_hbm.at[pl.ds(wid * IDX_ROWS, IDX_ROWS)], idx_v)

    issue_gather(0, 0)
    issue_gather(1, 1)

    def body(tt, carry):
        t0 = tt * 2
        t1 = t0 + 1
        wait_gather(0)
        issue_store(t0 - 2, 0)
        wait_gather(1)
        issue_store(t1 - 2, 1)
        wait_store(0)
        issue_gather(t0, 0)
        wait_store(1)
        issue_gather(t1, 1)
        return carry

    lax.fori_loop(1, N_ITERS // 2, body, 0)

    wait_gather(0)
    issue_store(N_ITERS - 2, 0)
    wait_gather(1)
    issue_store(N_ITERS - 1, 1)
    wait_store(0)
    wait_store(1)


def _untangle_body(g_ref, i_ref, eye_ref, o_ref):
    eye = eye_ref[...]
    for bb in range(B_BLKS):
        blk = g_ref[pl.ds(bb * 128, 128), :]
        t = lax.dot_general(blk, eye, TDIMS,
                            preferred_element_type=jnp.float32)
        row = i_ref[0, bb]
        hi = row >= OFFSET
        o_ref[0, :, bb] = jnp.where(hi[None, :], t[64:],
                                    t[:64]).reshape(8, 8, 128)


def kernel(x, W):
    idx = x.T.reshape(N // GATHER, GATHER).astype(jnp.int32)
    wt = W.T
    eye64 = jnp.eye(64, dtype=jnp.float32)
    eye128 = jnp.eye(128, dtype=jnp.float32)

    table = pl.pallas_call(
        _pack_body,
        grid=(PBLKS,),
        in_specs=[
            pl.BlockSpec((D, VB), lambda b: (0, b)),
            pl.BlockSpec((D, VB), lambda b: (0, PBLKS + b)),
            pl.BlockSpec((D, D), lambda b: (0, 0)),
        ],
        out_specs=pl.BlockSpec((VB, 128), lambda b: (b, 0)),
        out_shape=jax.ShapeDtypeStruct((OFFSET, 128), jnp.float32),
    )(wt, wt, eye64)

    mesh = plsc.VectorSubcoreMesh(core_axis_name="c", subcore_axis_name="s")
    run = pl.kernel(
        _gather_body,
        out_type=jax.ShapeDtypeStruct((N, 128), jnp.float32),
        mesh=mesh,
        scratch_types=[
            pltpu.VMEM((IDX_ROWS, GATHER), jnp.int32),
            pltpu.VMEM((2, GATHER), jnp.int32),
            pltpu.VMEM((2, GATHER, 128), jnp.float32),
            pltpu.SemaphoreType.DMA,
            pltpu.SemaphoreType.DMA,
            pltpu.SemaphoreType.DMA,
            pltpu.SemaphoreType.DMA,
        ],
        compiler_params=pltpu.CompilerParams(use_tc_tiling_on_sc=True),
    )
    g = run(idx, table)

    out5 = pl.pallas_call(
        _untangle_body,
        grid=(L,),
        in_specs=[
            pl.BlockSpec((B, 128), lambda l: (l, 0)),
            pl.BlockSpec((1, B_BLKS, 128), lambda l: (l, 0, 0)),
            pl.BlockSpec((128, 128), lambda l: (0, 0)),
        ],
        out_specs=pl.BlockSpec((1, 8, B_BLKS, 8, 128), lambda l: (l, 0, 0, 0, 0)),
        out_shape=jax.ShapeDtypeStruct((L, 8, B_BLKS, 8, 128), jnp.float32),
    )(g, idx.reshape(L, B_BLKS, 128), eye128)

    return out5.transpose(2, 4, 0, 1, 3).reshape(B, L, D)


# speedup vs baseline: 4.9477x; 1.4764x over previous
import jax
import jax.numpy as jnp
from jax import lax
from jax.experimental import pallas as pl
from jax.experimental.pallas import tpu as pltpu
from jax.experimental.pallas import tpu_sc as plsc

B, L, D = 4096, 200, 64
V = 1_000_000
B_BLKS = B // 128
N = B * L
NC, NS = 2, 16
NW = NC * NS
ROWS_PER_W = N // NW
GATHER = 128
N_ITERS = ROWS_PER_W // GATHER
IDX_ROWS = ROWS_PER_W // GATHER
VB = 4096
PBLKS = 123
OFFSET = PBLKS * VB
WT_BLKS = (V + VB - 1) // VB - 1
TDIMS = (((0,), (0,)), ((), ()))


def _pack_body(wa_ref, wb_ref, eye_ref, o_ref):
    ta = lax.dot_general(wa_ref[...], eye_ref[...], TDIMS,
                         preferred_element_type=jnp.float32)
    tb = lax.dot_general(wb_ref[...], eye_ref[...], TDIMS,
                         preferred_element_type=jnp.float32)
    o_ref[...] = jnp.concatenate([ta, tb], axis=1)


def _gather_body(idx_hbm, table_hbm, out_hbm, idx_v, sidx, rows_v,
                 g_sem0, g_sem1, s_sem0, s_sem1):
    wid = lax.axis_index("s") * NC + lax.axis_index("c")
    out_base = wid * ROWS_PER_W
    g_sems = (g_sem0, g_sem1)
    s_sems = (s_sem0, s_sem1)

    def make_sidx(t, buf):
        for k in range(8):
            v = idx_v[t, pl.ds(16 * k, 16)]
            sidx[buf, pl.ds(16 * k, 16)] = jnp.where(v >= OFFSET,
                                                     v - OFFSET, v)

    def issue_gather(t, buf):
        make_sidx(t, buf)
        pltpu.async_copy(table_hbm.at[sidx.at[buf]],
                         rows_v.at[buf], g_sems[buf])

    def wait_gather(buf):
        pltpu.make_async_copy(table_hbm.at[sidx.at[buf]],
                              rows_v.at[buf], g_sems[buf]).wait()

    def issue_store(t, buf):
        pltpu.async_copy(rows_v.at[buf],
                         out_hbm.at[pl.ds(out_base + t * GATHER, GATHER)],
                         s_sems[buf])

    def wait_store(buf):
        pltpu.make_async_copy(rows_v.at[buf],
                              out_hbm.at[pl.ds(out_base, GATHER)],
                              s_sems[buf]).wait()

    pltpu.sync_copy(idx_hbm.at[pl.ds(wid * IDX_ROWS, IDX_ROWS)], idx_v)

    issue_gather(0, 0)
    issue_gather(1, 1)

    def body(tt, carry):
        t0 = tt * 2
        t1 = t0 + 1
        wait_gather(0)
        issue_store(t0 - 2, 0)
        wait_gather(1)
        issue_store(t1 - 2, 1)
        wait_store(0)
        issue_gather(t0, 0)
        wait_store(1)
        issue_gather(t1, 1)
        return carry

    lax.fori_loop(1, N_ITERS // 2, body, 0)

    wait_gather(0)
    issue_store(N_ITERS - 2, 0)
    wait_gather(1)
    issue_store(N_ITERS - 1, 1)
    wait_store(0)
    wait_store(1)


def _untangle_body(g_ref, i_ref, eye_ref, o_ref):
    eye = eye_ref[...]
    for bb in range(B_BLKS):
        blk = g_ref[pl.ds(bb * 128, 128), :]
        t = lax.dot_general(blk, eye, TDIMS,
                            preferred_element_type=jnp.float32)
        row = i_ref[0, bb]
        hi = row >= OFFSET
        o_ref[0, :, bb] = jnp.where(hi[None, :], t[64:],
                                    t[:64]).reshape(8, 8, 128)


def kernel(x, W):
    idx = x.T.reshape(N // GATHER, GATHER).astype(jnp.int32)
    wt = W.T
    eye64 = jnp.eye(64, dtype=jnp.float32)
    eye128 = jnp.eye(128, dtype=jnp.float32)

    table = pl.pallas_call(
        _pack_body,
        grid=(PBLKS,),
        in_specs=[
            pl.BlockSpec((D, VB), lambda b: (0, b)),
            pl.BlockSpec((D, VB),
                         lambda b: (0, jnp.minimum(PBLKS + b, WT_BLKS))),
            pl.BlockSpec((D, D), lambda b: (0, 0)),
        ],
        out_specs=pl.BlockSpec((VB, 128), lambda b: (b, 0)),
        out_shape=jax.ShapeDtypeStruct((OFFSET, 128), jnp.float32),
    )(wt, wt, eye64)

    mesh = plsc.VectorSubcoreMesh(core_axis_name="c", subcore_axis_name="s")
    run = pl.kernel(
        _gather_body,
        out_type=jax.ShapeDtypeStruct((N, 128), jnp.float32),
        mesh=mesh,
        scratch_types=[
            pltpu.VMEM((IDX_ROWS, GATHER), jnp.int32),
            pltpu.VMEM((2, GATHER), jnp.int32),
            pltpu.VMEM((2, GATHER, 128), jnp.float32),
            pltpu.SemaphoreType.DMA,
            pltpu.SemaphoreType.DMA,
            pltpu.SemaphoreType.DMA,
            pltpu.SemaphoreType.DMA,
        ],
        compiler_params=pltpu.CompilerParams(use_tc_tiling_on_sc=True),
    )
    g = run(idx, table)

    out5 = pl.pallas_call(
        _untangle_body,
        grid=(L,),
        in_specs=[
            pl.BlockSpec((B, 128), lambda l: (l, 0)),
            pl.BlockSpec((1, B_BLKS, 128), lambda l: (l, 0, 0)),
            pl.BlockSpec((128, 128), lambda l: (0, 0)),
        ],
        out_specs=pl.BlockSpec((1, 8, B_BLKS, 8, 128), lambda l: (l, 0, 0, 0, 0)),
        out_shape=jax.ShapeDtypeStruct((L, 8, B_BLKS, 8, 128), jnp.float32),
    )(g, idx.reshape(L, B_BLKS, 128), eye128)

    return out5.transpose(2, 4, 0, 1, 3).reshape(B, L, D)


# speedup vs baseline: 5.2773x; 1.0666x over previous
import jax
import jax.numpy as jnp
from jax import lax
from jax.experimental import pallas as pl
from jax.experimental.pallas import tpu as pltpu
from jax.experimental.pallas import tpu_sc as plsc

B, L, D = 4096, 200, 64
V = 1_000_000
B_BLKS = B // 128
N = B * L
NC, NS = 2, 16
NW = NC * NS
ROWS_PER_W = N // NW
GATHER = 128
N_ITERS = ROWS_PER_W // GATHER
IDX_ROWS = ROWS_PER_W // GATHER
VB = 4096
PBLKS = 123
OFFSET = PBLKS * VB
WT_BLKS = (V + VB - 1) // VB - 1
TDIMS = (((0,), (0,)), ((), ()))


def _pack_body(wa_ref, wb_ref, eye_ref, o_ref):
    ta = lax.dot_general(wa_ref[...], eye_ref[...], TDIMS,
                         preferred_element_type=jnp.float32)
    tb = lax.dot_general(wb_ref[...], eye_ref[...], TDIMS,
                         preferred_element_type=jnp.float32)
    o_ref[...] = jnp.concatenate([ta, tb], axis=1)


def _gather_body(idx_hbm, table_hbm, out_hbm, idx_v, sidx, rows_v,
                 g_sem0, g_sem1, s_sem0, s_sem1):
    wid = lax.axis_index("s") * NC + lax.axis_index("c")
    out_base = wid * ROWS_PER_W
    g_sems = (g_sem0, g_sem1)
    s_sems = (s_sem0, s_sem1)

    def make_sidx(t, buf):
        for k in range(8):
            v = idx_v[t, pl.ds(16 * k, 16)]
            sidx[buf, pl.ds(16 * k, 16)] = jnp.where(v >= OFFSET,
                                                     v - OFFSET, v)

    def issue_gather(t, buf):
        make_sidx(t, buf)
        pltpu.async_copy(table_hbm.at[sidx.at[buf]],
                         rows_v.at[buf], g_sems[buf])

    def wait_gather(buf):
        pltpu.make_async_copy(table_hbm.at[sidx.at[buf]],
                              rows_v.at[buf], g_sems[buf]).wait()

    def issue_store(t, buf):
        pltpu.async_copy(rows_v.at[buf],
                         out_hbm.at[pl.ds(out_base + t * GATHER, GATHER)],
                         s_sems[buf])

    def wait_store(buf):
        pltpu.make_async_copy(rows_v.at[buf],
                              out_hbm.at[pl.ds(out_base, GATHER)],
                              s_sems[buf]).wait()

    pltpu.sync_copy(idx_hbm.at[pl.ds(wid * IDX_ROWS, IDX_ROWS)], idx_v)

    issue_gather(0, 0)
    issue_gather(1, 1)

    def body(tt, carry):
        t0 = tt * 2
        t1 = t0 + 1
        wait_gather(0)
        issue_store(t0 - 2, 0)
        wait_gather(1)
        issue_store(t1 - 2, 1)
        wait_store(0)
        issue_gather(t0, 0)
        wait_store(1)
        issue_gather(t1, 1)
        return carry

    lax.fori_loop(1, N_ITERS // 2, body, 0)

    wait_gather(0)
    issue_store(N_ITERS - 2, 0)
    wait_gather(1)
    issue_store(N_ITERS - 1, 1)
    wait_store(0)
    wait_store(1)


LU = 4


def _untangle_body(g_ref, i_ref, eye_ref, o_ref):
    eye = eye_ref[...]
    for ll in range(LU):
        for bb in range(B_BLKS):
            blk = g_ref[pl.ds((ll * B_BLKS + bb) * 128, 128), :]
            t = lax.dot_general(blk, eye, TDIMS,
                                preferred_element_type=jnp.float32)
            row = i_ref[ll, bb]
            hi = row >= OFFSET
            o_ref[ll, :, bb] = jnp.where(hi[None, :], t[64:],
                                         t[:64]).reshape(8, 8, 128)


def kernel(x, W):
    idx = x.T.reshape(N // GATHER, GATHER).astype(jnp.int32)
    wt = W.T
    eye64 = jnp.eye(64, dtype=jnp.float32)
    eye128 = jnp.eye(128, dtype=jnp.float32)

    table = pl.pallas_call(
        _pack_body,
        grid=(PBLKS,),
        in_specs=[
            pl.BlockSpec((D, VB), lambda b: (0, b)),
            pl.BlockSpec((D, VB),
                         lambda b: (0, jnp.minimum(PBLKS + b, WT_BLKS))),
            pl.BlockSpec((D, D), lambda b: (0, 0)),
        ],
        out_specs=pl.BlockSpec((VB, 128), lambda b: (b, 0)),
        out_shape=jax.ShapeDtypeStruct((OFFSET, 128), jnp.float32),
    )(wt, wt, eye64)

    mesh = plsc.VectorSubcoreMesh(core_axis_name="c", subcore_axis_name="s")
    run = pl.kernel(
        _gather_body,
        out_type=jax.ShapeDtypeStruct((N, 128), jnp.float32),
        mesh=mesh,
        scratch_types=[
            pltpu.VMEM((IDX_ROWS, GATHER), jnp.int32),
            pltpu.VMEM((2, GATHER), jnp.int32),
            pltpu.VMEM((2, GATHER, 128), jnp.float32),
            pltpu.SemaphoreType.DMA,
            pltpu.SemaphoreType.DMA,
            pltpu.SemaphoreType.DMA,
            pltpu.SemaphoreType.DMA,
        ],
        compiler_params=pltpu.CompilerParams(use_tc_tiling_on_sc=True),
    )
    g = run(idx, table)

    out5 = pl.pallas_call(
        _untangle_body,
        grid=(L // LU,),
        in_specs=[
            pl.BlockSpec((LU * B, 128), lambda l: (l, 0)),
            pl.BlockSpec((LU, B_BLKS, 128), lambda l: (l, 0, 0)),
            pl.BlockSpec((128, 128), lambda l: (0, 0)),
        ],
        out_specs=pl.BlockSpec((LU, 8, B_BLKS, 8, 128),
                               lambda l: (l, 0, 0, 0, 0)),
        out_shape=jax.ShapeDtypeStruct((L, 8, B_BLKS, 8, 128), jnp.float32),
    )(g, idx.reshape(L, B_BLKS, 128), eye128)

    return out5.transpose(2, 4, 0, 1, 3).reshape(B, L, D)
